# baseline (speedup 1.0000x reference)
import functools

import jax
import jax.numpy as jnp
from jax import lax
from jax.experimental import pallas as pl
from jax.experimental.pallas import tpu as pltpu
from jax.experimental.pallas import tpu_sc as plsc

_B = 64
_D = 32
_T = 1024
_K = 1024
_N = _B * _T
_COMMIT = 10.0
_BB = 8

_NC = 1
_NS = 16
_L = 16
_NW = _NC * _NS
_BPW = _B // _NW
_HCHUNK = _N // _NS
_DP = _D + 1


def _tc_body(x_ref, e_ref, idx_ref, loss_ref, loss_acc):
    b = pl.program_id(0)

    E = e_ref[...]
    e2 = jnp.sum(E * E, axis=1, keepdims=True)

    part = 0.0
    for i in range(_BB):
        X = x_ref[i]
        S = jax.lax.dot_general(E, X, (((1,), (0,)), ((), ())),
                                preferred_element_type=jnp.float32)
        S = S - 0.5 * e2

        m = jnp.max(S, axis=0)
        idx = jnp.argmax(S, axis=0)
        idx_ref[i, 0] = idx

        part = part + (jnp.sum(X * X) - 2.0 * jnp.sum(m))

    @pl.when(b == 0)
    def _init():
        loss_acc[0, 0] = 0.0

    loss_acc[0, 0] += part

    @pl.when(b == pl.num_programs(0) - 1)
    def _fini():
        loss_ref[0, 0] = loss_acc[0, 0] * ((1.0 + _COMMIT) / float(_N * _D))


def _sc_body(e_hbm, idx_hbm, out_hbm, usage_hbm,
             e_v, idx_v, hidx_v, ones_v, zeros_v, stage_v, hcnt_v, u_v,
             hist_sh):
    c = lax.axis_index("c")
    s = lax.axis_index("s")
    wid = c * _NS + s

    pltpu.sync_copy(e_hbm, e_v)
    pltpu.sync_copy(idx_hbm.at[pl.ds(wid * (_BPW * _T), _BPW * _T)], idx_v)
    pltpu.sync_copy(idx_hbm.at[pl.ds(s * _HCHUNK, _HCHUNK)], hidx_v)

    @pl.when(s == 0)
    def _zero_hist():
        @plsc.parallel_loop(0, _K // _L, unroll=4)
        def zb(j):
            zeros_v[pl.ds(pl.multiple_of(j * _L, _L), _L)] = (
                jnp.zeros((_L,), jnp.float32))
        pltpu.sync_copy(zeros_v, hist_sh)

    @plsc.parallel_loop(0, _HCHUNK // _L, unroll=4)
    def ob(j):
        ones_v[pl.ds(pl.multiple_of(j * _L, _L), _L)] = (
            jnp.ones((_L,), jnp.float32))

    plsc.subcore_barrier()

    pltpu.sync_copy(ones_v, hist_sh.at[hidx_v], add=True)

    for bi in range(_BPW):
        @plsc.parallel_loop(0, _T // _L, unroll=4)
        def gb(g):
            t0 = pl.multiple_of(bi * _T + g * _L, _L)
            base = idx_v[pl.ds(t0, _L)] * _DP
            for d in range(_D):
                vals = plsc.load_gather(e_v, [base + d])
                stage_v[d, pl.ds(pl.multiple_of(g * _L, _L), _L)] = vals
        pltpu.sync_copy(stage_v, out_hbm.at[wid * _BPW + bi])

    plsc.subcore_barrier()

    @pl.when(jnp.logical_and(s == 0, c == 0))
    def _usage():
        pltpu.sync_copy(hist_sh, hcnt_v)

        def ub(j, acc):
            v = hcnt_v[pl.ds(pl.multiple_of(j * _L, _L), _L)]
            return acc + jnp.where(v > 0.0, 1.0, 0.0)
        acc = lax.fori_loop(0, _K // _L, ub, jnp.zeros((_L,), jnp.float32))
        total = jnp.sum(acc).astype(jnp.int32)
        u_v[...] = jax.lax.broadcast_in_dim(total, (_L,), ())
        pltpu.sync_copy(u_v, usage_hbm)


_sc_call = functools.partial(
    pl.kernel,
    out_type=[
        jax.ShapeDtypeStruct((_B, _D, _T), jnp.float32),
        jax.ShapeDtypeStruct((_L,), jnp.int32),
    ],
    mesh=plsc.VectorSubcoreMesh(core_axis_name="c", subcore_axis_name="s",
                                num_cores=_NC, num_subcores=_NS),
    compiler_params=pltpu.CompilerParams(needs_layout_passes=False),
    scratch_types=[
        pltpu.VMEM((_K * _DP,), jnp.float32),
        pltpu.VMEM((_BPW * _T,), jnp.int32),
        pltpu.VMEM((_HCHUNK,), jnp.int32),
        pltpu.VMEM((_HCHUNK,), jnp.float32),
        pltpu.VMEM((_K,), jnp.float32),
        pltpu.VMEM((_D, _T), jnp.float32),
        pltpu.VMEM((_K,), jnp.float32),
        pltpu.VMEM((_L,), jnp.int32),
        pltpu.VMEM_SHARED((_K,), jnp.float32),
    ],
)(_sc_body)


def kernel(inputs, embedding_weight):
    idx3, loss = pl.pallas_call(
        _tc_body,
        grid=(_B // _BB,),
        in_specs=[
            pl.BlockSpec((_BB, _D, _T), lambda b: (b, 0, 0)),
            pl.BlockSpec((_K, _D), lambda b: (0, 0)),
        ],
        out_specs=[
            pl.BlockSpec((_BB, 1, _T), lambda b: (b, 0, 0)),
            pl.BlockSpec(memory_space=pltpu.SMEM),
        ],
        out_shape=[
            jax.ShapeDtypeStruct((_B, 1, _T), jnp.int32),
            jax.ShapeDtypeStruct((1, 1), jnp.float32),
        ],
        scratch_shapes=[
            pltpu.SMEM((1, 1), jnp.float32),
        ],
    )(inputs, embedding_weight)

    e_pad = jnp.concatenate(
        [embedding_weight, jnp.zeros((_K, 1), jnp.float32)], axis=1)
    out, usage = _sc_call(e_pad.reshape(_K * _DP), idx3.reshape(_N))
    return (out, loss[0, 0], usage[0], idx3.reshape(_N, 1))


# speedup vs baseline: 1.0387x; 1.0387x over previous
import functools

import jax
import jax.numpy as jnp
from jax import lax
from jax.experimental import pallas as pl
from jax.experimental.pallas import tpu as pltpu
from jax.experimental.pallas import tpu_sc as plsc

_B = 64
_D = 32
_T = 1024
_K = 1024
_N = _B * _T
_COMMIT = 10.0
_BB = 8

_NC = 2
_NS = 16
_L = 16
_NW = _NC * _NS
_BPW = _B // _NW
_HCHUNK = _N // _NS
_DP = _D + 1


def _tc_body(x_ref, e_ref, idx_ref, loss_ref, loss_acc):
    b = pl.program_id(0)

    E = e_ref[...]
    e2 = jnp.sum(E * E, axis=1, keepdims=True)

    part = 0.0
    for i in range(_BB):
        X = x_ref[i]
        S = jax.lax.dot_general(E, X, (((1,), (0,)), ((), ())),
                                preferred_element_type=jnp.float32)
        S = S - 0.5 * e2

        m = jnp.max(S, axis=0)
        idx = jnp.argmax(S, axis=0)
        idx_ref[i, 0] = idx

        part = part + (jnp.sum(X * X) - 2.0 * jnp.sum(m))

    @pl.when(b == 0)
    def _init():
        loss_acc[0, 0] = 0.0

    loss_acc[0, 0] += part

    @pl.when(b == pl.num_programs(0) - 1)
    def _fini():
        loss_ref[0, 0] = loss_acc[0, 0] * ((1.0 + _COMMIT) / float(_N * _D))


def _sc_body(e_hbm, idx_hbm, out_hbm, usage_hbm,
             e_v, idx_v, hidx_v, ones_v, zeros_v, stage_v, hcnt_v, u_v,
             hist_sh):
    c = lax.axis_index("c")
    s = lax.axis_index("s")
    wid = c * _NS + s

    pltpu.sync_copy(e_hbm, e_v)
    pltpu.sync_copy(idx_hbm.at[pl.ds(wid * (_BPW * _T), _BPW * _T)], idx_v)
    pltpu.sync_copy(idx_hbm.at[pl.ds(s * _HCHUNK, _HCHUNK)], hidx_v)

    @pl.when(s == 0)
    def _zero_hist():
        @plsc.parallel_loop(0, _K // _L, unroll=4)
        def zb(j):
            zeros_v[pl.ds(pl.multiple_of(j * _L, _L), _L)] = (
                jnp.zeros((_L,), jnp.float32))
        pltpu.sync_copy(zeros_v, hist_sh)

    @plsc.parallel_loop(0, _HCHUNK // _L, unroll=4)
    def ob(j):
        ones_v[pl.ds(pl.multiple_of(j * _L, _L), _L)] = (
            jnp.ones((_L,), jnp.float32))

    plsc.subcore_barrier()

    pltpu.sync_copy(ones_v, hist_sh.at[hidx_v], add=True)

    for bi in range(_BPW):
        @plsc.parallel_loop(0, _T // _L, unroll=4)
        def gb(g):
            t0 = pl.multiple_of(bi * _T + g * _L, _L)
            base = idx_v[pl.ds(t0, _L)] * _DP
            for d in range(_D):
                vals = plsc.load_gather(e_v, [base + d])
                stage_v[d, pl.ds(pl.multiple_of(g * _L, _L), _L)] = vals
        pltpu.sync_copy(stage_v, out_hbm.at[wid * _BPW + bi])

    plsc.subcore_barrier()

    @pl.when(jnp.logical_and(s == 0, c == 0))
    def _usage():
        pltpu.sync_copy(hist_sh, hcnt_v)

        def ub(j, acc):
            v = hcnt_v[pl.ds(pl.multiple_of(j * _L, _L), _L)]
            return acc + jnp.where(v > 0.0, 1.0, 0.0)
        acc = lax.fori_loop(0, _K // _L, ub, jnp.zeros((_L,), jnp.float32))
        total = jnp.sum(acc).astype(jnp.int32)
        u_v[...] = jax.lax.broadcast_in_dim(total, (_L,), ())
        pltpu.sync_copy(u_v, usage_hbm)


_sc_call = functools.partial(
    pl.kernel,
    out_type=[
        jax.ShapeDtypeStruct((_B, _D, _T), jnp.float32),
        jax.ShapeDtypeStruct((_L,), jnp.int32),
    ],
    mesh=plsc.VectorSubcoreMesh(core_axis_name="c", subcore_axis_name="s",
                                num_cores=_NC, num_subcores=_NS),
    compiler_params=pltpu.CompilerParams(needs_layout_passes=False),
    scratch_types=[
        pltpu.VMEM((_K * _DP,), jnp.float32),
        pltpu.VMEM((_BPW * _T,), jnp.int32),
        pltpu.VMEM((_HCHUNK,), jnp.int32),
        pltpu.VMEM((_HCHUNK,), jnp.float32),
        pltpu.VMEM((_K,), jnp.float32),
        pltpu.VMEM((_D, _T), jnp.float32),
        pltpu.VMEM((_K,), jnp.float32),
        pltpu.VMEM((_L,), jnp.int32),
        pltpu.VMEM_SHARED((_K,), jnp.float32),
    ],
)(_sc_body)


def kernel(inputs, embedding_weight):
    idx3, loss = pl.pallas_call(
        _tc_body,
        grid=(_B // _BB,),
        in_specs=[
            pl.BlockSpec((_BB, _D, _T), lambda b: (b, 0, 0)),
            pl.BlockSpec((_K, _D), lambda b: (0, 0)),
        ],
        out_specs=[
            pl.BlockSpec((_BB, 1, _T), lambda b: (b, 0, 0)),
            pl.BlockSpec(memory_space=pltpu.SMEM),
        ],
        out_shape=[
            jax.ShapeDtypeStruct((_B, 1, _T), jnp.int32),
            jax.ShapeDtypeStruct((1, 1), jnp.float32),
        ],
        scratch_shapes=[
            pltpu.SMEM((1, 1), jnp.float32),
        ],
    )(inputs, embedding_weight)

    e_pad = jnp.concatenate(
        [embedding_weight, jnp.zeros((_K, 1), jnp.float32)], axis=1)
    out, usage = _sc_call(e_pad.reshape(_K * _DP), idx3.reshape(_N))
    return (out, loss[0, 0], usage[0], idx3.reshape(_N, 1))
